# baseline (speedup 1.0000x reference)
import functools

import jax
import jax.numpy as jnp
from jax import lax
from jax.experimental import pallas as pl
from jax.experimental.pallas import tpu as pltpu
from jax.experimental.pallas import tpu_sc as plsc

N = 16384
C = 1000
NC = 2
NS = 16
NW = NC * NS
ROWS_PER_W = N // NW
BLK = 128
NBLK = ROWS_PER_W // BLK
L = 16


def _sc_one_hot(x_hbm, z_hbm, out_hbm, zbuf, xv, idxv, ones, sem_z, sem_s):
    wid = lax.axis_index("s") * NC + lax.axis_index("c")
    base = wid * ROWS_PER_W

    pltpu.sync_copy(z_hbm, zbuf)
    pltpu.sync_copy(x_hbm.at[pl.ds(base, ROWS_PER_W)], xv)

    one_vec = jnp.ones((L,), jnp.float32)
    for g in range(BLK // L):
        ones[pl.ds(g * L, L)] = one_vec

    lane = lax.iota(jnp.int32, L)
    for g in range(ROWS_PER_W // L):
        rows = (base + g * L) + lane
        vals = rows * C + xv[pl.ds(g * L, L)]
        idxv[g // (BLK // L), pl.ds((g % (BLK // L)) * L, L)] = vals

    zcopies = []
    for c in range(NBLK):
        dst = out_hbm.at[pl.ds((base + c * BLK) * C, BLK * C)]
        cp = pltpu.make_async_copy(zbuf, dst, sem_z)
        cp.start()
        zcopies.append(cp)
    for cp in zcopies:
        cp.wait()

    scopies = []
    for j in range(NBLK):
        cp = pltpu.make_async_copy(ones, out_hbm.at[idxv.at[j]], sem_s)
        cp.start()
        scopies.append(cp)
    for cp in scopies:
        cp.wait()


@jax.jit
def kernel(x):
    mesh = plsc.VectorSubcoreMesh(core_axis_name="c", subcore_axis_name="s")
    call = functools.partial(
        pl.kernel,
        out_type=jax.ShapeDtypeStruct((N * C,), jnp.float32),
        mesh=mesh,
        scratch_types=[
            pltpu.VMEM((BLK * C,), jnp.float32),
            pltpu.VMEM((ROWS_PER_W,), jnp.int32),
            pltpu.VMEM((NBLK, BLK), jnp.int32),
            pltpu.VMEM((BLK,), jnp.float32),
            pltpu.SemaphoreType.DMA,
            pltpu.SemaphoreType.DMA,
        ],
    )(_sc_one_hot)
    z = jnp.zeros((BLK * C,), jnp.float32)
    flat = call(x.astype(jnp.int32), z)
    return flat.reshape(N, C)


# speedup vs baseline: 1.7052x; 1.7052x over previous
import functools

import jax
import jax.numpy as jnp
from jax import lax
from jax.experimental import pallas as pl
from jax.experimental.pallas import tpu as pltpu
from jax.experimental.pallas import tpu_sc as plsc

N = 16384
C = 1000
NC = 2
NS = 16
NW = NC * NS
ROWS_PER_W = N // NW
BLK = 32
NBLK = ROWS_PER_W // BLK
L = 16
GRP = BLK // L


def _sc_one_hot(x_hbm, z_hbm, out_hbm, zb0, zb1, xv, sem0, sem1, semx):
    wid = lax.axis_index("s") * NC + lax.axis_index("c")
    base = wid * ROWS_PER_W

    cpx = pltpu.make_async_copy(x_hbm.at[pl.ds(base, ROWS_PER_W)], xv, semx)
    cpx.start()
    cpz0 = pltpu.make_async_copy(z_hbm, zb0, sem0)
    cpz1 = pltpu.make_async_copy(z_hbm, zb1, sem1)
    cpz0.start()
    cpz1.start()
    cpx.wait()
    cpz0.wait()
    cpz1.wait()

    bufs = (zb0, zb1)
    sems = (sem0, sem1)
    lane = lax.iota(jnp.int32, L)
    ones = jnp.ones((L,), jnp.float32)
    zeros = jnp.zeros((L,), jnp.float32)

    copies = [None, None]
    for b in range(NBLK):
        B = b % 2
        buf = bufs[B]
        if copies[B] is not None:
            copies[B].wait()
            pb = b - 2
            for g in range(GRP):
                cols = xv[pl.ds(pb * BLK + g * L, L)]
                plsc.store_scatter(buf, [g * L + lane, cols], zeros)
        for g in range(GRP):
            cols = xv[pl.ds(b * BLK + g * L, L)]
            plsc.store_scatter(buf, [g * L + lane, cols], ones)
        cp = pltpu.make_async_copy(
            buf, out_hbm.at[pl.ds(base + b * BLK, BLK)], sems[B]
        )
        cp.start()
        copies[B] = cp
    copies[0].wait()
    copies[1].wait()


@jax.jit
def kernel(x):
    mesh = plsc.VectorSubcoreMesh(core_axis_name="c", subcore_axis_name="s")
    call = functools.partial(
        pl.kernel,
        out_type=jax.ShapeDtypeStruct((N, C), jnp.float32),
        mesh=mesh,
        compiler_params=pltpu.CompilerParams(needs_layout_passes=False),
        scratch_types=[
            pltpu.VMEM((BLK, C), jnp.float32),
            pltpu.VMEM((BLK, C), jnp.float32),
            pltpu.VMEM((ROWS_PER_W,), jnp.int32),
            pltpu.SemaphoreType.DMA,
            pltpu.SemaphoreType.DMA,
            pltpu.SemaphoreType.DMA,
        ],
    )(_sc_one_hot)
    z = jnp.zeros((BLK, C), jnp.float32)
    return call(x.astype(jnp.int32), z)


# speedup vs baseline: 3.4650x; 2.0320x over previous
import functools

import jax
import jax.numpy as jnp
from jax import lax
from jax.experimental import pallas as pl
from jax.experimental.pallas import tpu as pltpu
from jax.experimental.pallas import tpu_sc as plsc

N = 16384
C = 1000
NC = 2
NS = 16
NW = NC * NS
COLS_PER_W = N // NW
BLK = 128
NBLK = COLS_PER_W // BLK
L = 16
GRP = BLK // L


def _sc_one_hot_t(x_hbm, z_hbm, out_hbm, buf, xv, semz, semx, semo):
    wid = lax.axis_index("s") * NC + lax.axis_index("c")
    base = wid * COLS_PER_W

    cpz = pltpu.make_async_copy(z_hbm, buf, semz)
    cpz.start()
    cpx = pltpu.make_async_copy(x_hbm.at[pl.ds(base, COLS_PER_W)], xv, semx)
    cpx.start()
    cpx.wait()
    cpz.wait()

    lane = lax.iota(jnp.int32, L)
    ones = jnp.ones((L,), jnp.float32)
    zeros = jnp.zeros((L,), jnp.float32)

    for b in range(NBLK):
        for g in range(GRP):
            rows = xv[pl.ds(b * BLK + g * L, L)]
            plsc.store_scatter(buf, [rows, g * L + lane], ones)
        cp = pltpu.make_async_copy(
            buf, out_hbm.at[:, pl.ds(base + b * BLK, BLK)], semo
        )
        cp.start()
        cp.wait()
        if b + 1 < NBLK:
            for g in range(GRP):
                rows = xv[pl.ds(b * BLK + g * L, L)]
                plsc.store_scatter(buf, [rows, g * L + lane], zeros)


@jax.jit
def kernel(x):
    mesh = plsc.VectorSubcoreMesh(core_axis_name="c", subcore_axis_name="s")
    call = functools.partial(
        pl.kernel,
        out_type=jax.ShapeDtypeStruct((C, N), jnp.float32),
        mesh=mesh,
        compiler_params=pltpu.CompilerParams(needs_layout_passes=False),
        scratch_types=[
            pltpu.VMEM((C, BLK), jnp.float32),
            pltpu.VMEM((COLS_PER_W,), jnp.int32),
            pltpu.SemaphoreType.DMA,
            pltpu.SemaphoreType.DMA,
            pltpu.SemaphoreType.DMA,
        ],
    )(_sc_one_hot_t)
    z = jnp.zeros((C, BLK), jnp.float32)
    out_t = call(x.astype(jnp.int32), z)
    return out_t.T


# speedup vs baseline: 3.5141x; 1.0142x over previous
import functools

import jax
import jax.numpy as jnp
from jax import lax
from jax.experimental import pallas as pl
from jax.experimental.pallas import tpu as pltpu
from jax.experimental.pallas import tpu_sc as plsc

N = 16384
C = 1000
NC = 2
NS = 16
NW = NC * NS
COLS_PER_W = N // NW
BLK = 128
NBLK = COLS_PER_W // BLK
L = 16
GRP = BLK // L


def _sc_one_hot_t(x_hbm, z_hbm, out_hbm, buf, xv, semz, semx, semo):
    wid = lax.axis_index("s") * NC + lax.axis_index("c")
    base = wid * COLS_PER_W

    cpz = pltpu.make_async_copy(z_hbm, buf, semz)
    cpz.start()
    cpx = pltpu.make_async_copy(x_hbm.at[pl.ds(base, COLS_PER_W)], xv, semx)
    cpx.start()
    cpx.wait()
    cpz.wait()

    lane = lax.iota(jnp.int32, L)
    ones = jnp.ones((L,), jnp.float32)
    zeros = jnp.zeros((L,), jnp.float32)

    for b in range(NBLK):
        for g in range(GRP):
            rows = xv[pl.ds(b * BLK + g * L, L)]
            plsc.store_scatter(buf, [rows, g * L + lane], ones)
        cp = pltpu.make_async_copy(
            buf, out_hbm.at[:, pl.ds(base + b * BLK, BLK)], semo
        )
        cp.start()
        cp.wait()
        if b + 1 < NBLK:
            for g in range(GRP):
                rows = xv[pl.ds(b * BLK + g * L, L)]
                plsc.store_scatter(buf, [rows, g * L + lane], zeros)


@jax.jit
def kernel(x):
    mesh = plsc.VectorSubcoreMesh(core_axis_name="c", subcore_axis_name="s")
    call = functools.partial(
        pl.kernel,
        out_type=jax.ShapeDtypeStruct((C, N), jnp.float32),
        mesh=mesh,
        compiler_params=pltpu.CompilerParams(
            needs_layout_passes=False,
            skip_device_barrier=True,
            disable_bounds_checks=True,
            disable_semaphore_checks=True,
        ),
        scratch_types=[
            pltpu.VMEM((C, BLK), jnp.float32),
            pltpu.VMEM((COLS_PER_W,), jnp.int32),
            pltpu.SemaphoreType.DMA,
            pltpu.SemaphoreType.DMA,
            pltpu.SemaphoreType.DMA,
        ],
    )(_sc_one_hot_t)
    z = jnp.zeros((C, BLK), jnp.float32)
    out_t = call(x.astype(jnp.int32), z)
    return out_t.T
